# baseline (device time: 626971 ns/iter reference)
import os

import jax
import jax.numpy as jnp
from jax import lax
from jax.experimental import pallas as pl
from jax.experimental.pallas import tpu as pltpu

N_DEV = 4
SQ = 256
D = 1024
SKV = 4096
H_PER = 8
DH = 128
SCALE = 0.08838834764831843

BF16 = jnp.bfloat16
F32 = jnp.float32

PROBE = os.environ.get("KERNEL_PROBE", "")


def kernel(x, Wq, Wo, K_ext, V_ext):
    K_ext = K_ext.reshape(N_DEV, SKV, 32 * DH)
    V_ext = V_ext.reshape(N_DEV, SKV, 32 * DH)

    def body(x_ref, wq_ref, wo_ref, k_hbm, v_hbm, out_ref,
             xall, wqb, wob, kbuf, vbuf, obuf, ps_buf, pr_buf,
             ag_send, ag_recv, ps_send, pr_recv, kv_sems):
        i = lax.axis_index("i")

        barrier_sem = pltpu.get_barrier_semaphore()
        for d in range(1, N_DEV):
            pl.semaphore_signal(
                barrier_sem, inc=1,
                device_id=((i + d) % N_DEV,),
                device_id_type=pl.DeviceIdType.MESH,
            )
        pl.semaphore_wait(barrier_sem, N_DEV - 1)

        xall[0] = x_ref[0].astype(BF16)
        ag_rdmas = []
        for d in range(1, N_DEV):
            r = pltpu.make_async_remote_copy(
                src_ref=xall.at[0],
                dst_ref=xall.at[N_DEV - d],
                send_sem=ag_send.at[d - 1],
                recv_sem=ag_recv.at[N_DEV - d],
                device_id=((i + d) % N_DEV,),
                device_id_type=pl.DeviceIdType.MESH,
            )
            r.start()
            ag_rdmas.append(r)

        wqb[...] = wq_ref[...].astype(BF16)
        wob[...] = wo_ref[...].astype(BF16)

        h0c = i * (H_PER * DH)

        GW = (H_PER // 2) * DH
        N_CHUNK = 2 * N_DEV

        def start_kv(c):
            d, g = divmod(c, 2)
            b = (i + d) % N_DEV
            slot = c % 2
            ck = pltpu.make_async_copy(
                k_hbm.at[b, :, pl.ds(h0c + g * GW, GW)], kbuf.at[slot],
                kv_sems.at[slot, 0])
            cv = pltpu.make_async_copy(
                v_hbm.at[b, :, pl.ds(h0c + g * GW, GW)], vbuf.at[slot],
                kv_sems.at[slot, 1])
            ck.start()
            cv.start()
            return ck, cv

        if PROBE == "nodma":
            kv_inflight = {}
        else:
            kv_inflight = {c: start_kv(c) for c in range(2)}

        send_rdmas = []
        acc = None
        for d in range(N_DEV):
            b = (i + d) % N_DEV

            if d > 0:
                recv = pltpu.make_async_remote_copy(
                    src_ref=xall.at[0], dst_ref=xall.at[d],
                    send_sem=ag_send.at[0], recv_sem=ag_recv.at[d],
                    device_id=(i,), device_id_type=pl.DeviceIdType.MESH,
                )
                recv.wait_recv()

            q = jnp.dot(xall[d], wqb[...], preferred_element_type=F32)
            q = q.astype(BF16)

            for g in range(2):
                c = 2 * d + g
                slot = c % 2
                if PROBE != "nodma":
                    ck, cv = kv_inflight.pop(c)
                    ck.wait()
                    cv.wait()
                for h in range(H_PER // 2):
                    hh = g * (H_PER // 2) + h
                    qh = q[:, hh * DH:(hh + 1) * DH]
                    if PROBE == "nocompute":
                        obuf[:, hh * DH:(hh + 1) * DH] = (
                            kbuf[slot, :SQ, h * DH:(h + 1) * DH]
                            + vbuf[slot, :SQ, h * DH:(h + 1) * DH]
                        ).astype(BF16)
                    else:
                        kh = kbuf[slot, :, h * DH:(h + 1) * DH].astype(BF16)
                        s = lax.dot_general(
                            qh, kh, (((1,), (1,)), ((), ())),
                            preferred_element_type=F32) * SCALE
                        m = jnp.max(s, axis=1, keepdims=True)
                        p = jnp.exp(s - m)
                        l = jnp.sum(p, axis=1, keepdims=True)
                        vh = vbuf[slot, :, h * DH:(h + 1) * DH].astype(BF16)
                        oh = lax.dot_general(
                            p.astype(BF16), vh, (((1,), (0,)), ((), ())),
                            preferred_element_type=F32)
                        obuf[:, hh * DH:(hh + 1) * DH] = (
                            oh / l).astype(BF16)

                if PROBE != "nodma" and c + 2 < N_CHUNK:
                    kv_inflight[c + 2] = start_kv(c + 2)

            part = jnp.dot(obuf[...], wob[...], preferred_element_type=F32)

            if d == 0:
                acc = part
            else:
                ps_buf[d - 1] = part.astype(BF16)
                r = pltpu.make_async_remote_copy(
                    src_ref=ps_buf.at[d - 1],
                    dst_ref=pr_buf.at[N_DEV - d],
                    send_sem=ps_send.at[d - 1],
                    recv_sem=pr_recv.at[N_DEV - d],
                    device_id=(b,), device_id_type=pl.DeviceIdType.MESH,
                )
                r.start()
                send_rdmas.append(r)

        for s in range(1, N_DEV):
            recv = pltpu.make_async_remote_copy(
                src_ref=ps_buf.at[0], dst_ref=pr_buf.at[s],
                send_sem=ps_send.at[0], recv_sem=pr_recv.at[s],
                device_id=(i,), device_id_type=pl.DeviceIdType.MESH,
            )
            recv.wait_recv()
            acc = acc + pr_buf[s].astype(F32)

        for r in ag_rdmas:
            r.wait_send()
        for r in send_rdmas:
            r.wait_send()

        out_ref[0] = acc

    return pl.pallas_call(
        body,
        out_shape=jax.ShapeDtypeStruct((1, SQ, D), F32),
        in_specs=[
            pl.BlockSpec(memory_space=pltpu.VMEM),
            pl.BlockSpec(memory_space=pltpu.VMEM),
            pl.BlockSpec(memory_space=pltpu.VMEM),
            pl.BlockSpec(memory_space=pl.ANY),
            pl.BlockSpec(memory_space=pl.ANY),
        ],
        out_specs=pl.BlockSpec(memory_space=pltpu.VMEM),
        scratch_shapes=[
            pltpu.VMEM((N_DEV, SQ, D), BF16),
            pltpu.VMEM((D, D), BF16),
            pltpu.VMEM((D, D), BF16),
            pltpu.VMEM((2, SKV, (H_PER // 2) * DH), F32),
            pltpu.VMEM((2, SKV, (H_PER // 2) * DH), F32),
            pltpu.VMEM((SQ, D), BF16),
            pltpu.VMEM((N_DEV - 1, SQ, D), BF16),
            pltpu.VMEM((N_DEV, SQ, D), BF16),
            pltpu.SemaphoreType.DMA((N_DEV - 1,)),
            pltpu.SemaphoreType.DMA((N_DEV,)),
            pltpu.SemaphoreType.DMA((N_DEV - 1,)),
            pltpu.SemaphoreType.DMA((N_DEV,)),
            pltpu.SemaphoreType.DMA((2, 2)),
        ],
        compiler_params=pltpu.CompilerParams(
            collective_id=0,
            vmem_limit_bytes=63 * 1024 * 1024,
        ),
        interpret=(
            pltpu.InterpretParams()
            if os.environ.get("PL_INTERPRET") == "1"
            else False
        ),
    )(x, Wq, Wo, K_ext, V_ext)


# device time: 93881 ns/iter; 6.6784x vs baseline; 6.6784x over previous
import os

import jax
import jax.numpy as jnp
from jax import lax
from jax.experimental import pallas as pl
from jax.experimental.pallas import tpu as pltpu

N_DEV = 4
SQ = 256
D = 1024
SKV = 4096
H_PER = 8
DH = 128
SCALE = 0.08838834764831843

BF16 = jnp.bfloat16
F32 = jnp.float32

PROBE = os.environ.get("KERNEL_PROBE", "")


def kernel(x, Wq, Wo, K_ext, V_ext):
    def body(x_ref, wq_ref, wo_ref, k_hbm, v_hbm, out_ref,
             xall, wqb, wob, kbuf, vbuf, obuf, ps_buf, pr_buf,
             ag_send, ag_recv, ps_send, pr_recv, kv_sems):
        i = lax.axis_index("i")

        barrier_sem = pltpu.get_barrier_semaphore()
        for d in range(1, N_DEV):
            pl.semaphore_signal(
                barrier_sem, inc=1,
                device_id=((i + d) % N_DEV,),
                device_id_type=pl.DeviceIdType.MESH,
            )
        pl.semaphore_wait(barrier_sem, N_DEV - 1)

        xall[0] = x_ref[0].astype(BF16)
        ag_rdmas = []
        for d in range(1, N_DEV):
            r = pltpu.make_async_remote_copy(
                src_ref=xall.at[0],
                dst_ref=xall.at[N_DEV - d],
                send_sem=ag_send.at[d - 1],
                recv_sem=ag_recv.at[N_DEV - d],
                device_id=((i + d) % N_DEV,),
                device_id_type=pl.DeviceIdType.MESH,
            )
            r.start()
            ag_rdmas.append(r)

        wqb[...] = wq_ref[...].astype(BF16)
        wob[...] = wo_ref[...].astype(BF16)

        h0c = i * (H_PER * DH)

        k2 = k_hbm.reshape(N_DEV, SKV, 32 * DH)
        v2 = v_hbm.reshape(N_DEV, SKV, 32 * DH)

        GW = (H_PER // 2) * DH
        N_CHUNK = 2 * N_DEV

        def start_kv(c):
            d, g = divmod(c, 2)
            b = (i + d) % N_DEV
            slot = c % 2
            ck = pltpu.make_async_copy(
                k2.at[b, :, pl.ds(h0c + g * GW, GW)], kbuf.at[slot],
                kv_sems.at[slot, 0])
            cv = pltpu.make_async_copy(
                v2.at[b, :, pl.ds(h0c + g * GW, GW)], vbuf.at[slot],
                kv_sems.at[slot, 1])
            ck.start()
            cv.start()
            return ck, cv

        if PROBE == "nodma":
            kv_inflight = {}
        else:
            kv_inflight = {c: start_kv(c) for c in range(2)}

        send_rdmas = []
        acc = None
        for d in range(N_DEV):
            b = (i + d) % N_DEV

            if d > 0:
                recv = pltpu.make_async_remote_copy(
                    src_ref=xall.at[0], dst_ref=xall.at[d],
                    send_sem=ag_send.at[0], recv_sem=ag_recv.at[d],
                    device_id=(i,), device_id_type=pl.DeviceIdType.MESH,
                )
                recv.wait_recv()

            q = jnp.dot(xall[d], wqb[...], preferred_element_type=F32)
            q = q.astype(BF16)

            for g in range(2):
                c = 2 * d + g
                slot = c % 2
                if PROBE != "nodma":
                    ck, cv = kv_inflight.pop(c)
                    ck.wait()
                    cv.wait()
                for h in range(H_PER // 2):
                    hh = g * (H_PER // 2) + h
                    qh = q[:, hh * DH:(hh + 1) * DH]
                    if PROBE == "nocompute":
                        obuf[:, hh * DH:(hh + 1) * DH] = (
                            kbuf[slot, :SQ, h * DH:(h + 1) * DH]
                            + vbuf[slot, :SQ, h * DH:(h + 1) * DH]
                        ).astype(BF16)
                    else:
                        kh = kbuf[slot, :, h * DH:(h + 1) * DH].astype(BF16)
                        s = lax.dot_general(
                            qh, kh, (((1,), (1,)), ((), ())),
                            preferred_element_type=F32) * SCALE
                        m = jnp.max(s, axis=1, keepdims=True)
                        p = jnp.exp(s - m)
                        l = jnp.sum(p, axis=1, keepdims=True)
                        vh = vbuf[slot, :, h * DH:(h + 1) * DH].astype(BF16)
                        oh = lax.dot_general(
                            p.astype(BF16), vh, (((1,), (0,)), ((), ())),
                            preferred_element_type=F32)
                        obuf[:, hh * DH:(hh + 1) * DH] = (
                            oh / l).astype(BF16)

                if PROBE != "nodma" and c + 2 < N_CHUNK:
                    kv_inflight[c + 2] = start_kv(c + 2)

            part = jnp.dot(obuf[...], wob[...], preferred_element_type=F32)

            if d == 0:
                acc = part
            else:
                ps_buf[d - 1] = part.astype(BF16)
                r = pltpu.make_async_remote_copy(
                    src_ref=ps_buf.at[d - 1],
                    dst_ref=pr_buf.at[N_DEV - d],
                    send_sem=ps_send.at[d - 1],
                    recv_sem=pr_recv.at[N_DEV - d],
                    device_id=(b,), device_id_type=pl.DeviceIdType.MESH,
                )
                r.start()
                send_rdmas.append(r)

        for s in range(1, N_DEV):
            recv = pltpu.make_async_remote_copy(
                src_ref=ps_buf.at[0], dst_ref=pr_buf.at[s],
                send_sem=ps_send.at[0], recv_sem=pr_recv.at[s],
                device_id=(i,), device_id_type=pl.DeviceIdType.MESH,
            )
            recv.wait_recv()
            acc = acc + pr_buf[s].astype(F32)

        for r in ag_rdmas:
            r.wait_send()
        for r in send_rdmas:
            r.wait_send()

        out_ref[0] = acc

    return pl.pallas_call(
        body,
        out_shape=jax.ShapeDtypeStruct((1, SQ, D), F32),
        in_specs=[
            pl.BlockSpec(memory_space=pltpu.VMEM),
            pl.BlockSpec(memory_space=pltpu.VMEM),
            pl.BlockSpec(memory_space=pltpu.VMEM),
            pl.BlockSpec(memory_space=pl.ANY),
            pl.BlockSpec(memory_space=pl.ANY),
        ],
        out_specs=pl.BlockSpec(memory_space=pltpu.VMEM),
        scratch_shapes=[
            pltpu.VMEM((N_DEV, SQ, D), BF16),
            pltpu.VMEM((D, D), BF16),
            pltpu.VMEM((D, D), BF16),
            pltpu.VMEM((2, SKV, (H_PER // 2) * DH), F32),
            pltpu.VMEM((2, SKV, (H_PER // 2) * DH), F32),
            pltpu.VMEM((SQ, D), BF16),
            pltpu.VMEM((N_DEV - 1, SQ, D), BF16),
            pltpu.VMEM((N_DEV, SQ, D), BF16),
            pltpu.SemaphoreType.DMA((N_DEV - 1,)),
            pltpu.SemaphoreType.DMA((N_DEV,)),
            pltpu.SemaphoreType.DMA((N_DEV - 1,)),
            pltpu.SemaphoreType.DMA((N_DEV,)),
            pltpu.SemaphoreType.DMA((2, 2)),
        ],
        compiler_params=pltpu.CompilerParams(
            collective_id=0,
            vmem_limit_bytes=63 * 1024 * 1024,
        ),
        interpret=(
            pltpu.InterpretParams()
            if os.environ.get("PL_INTERPRET") == "1"
            else False
        ),
    )(x, Wq, Wo, K_ext, V_ext)


# device time: 93482 ns/iter; 6.7069x vs baseline; 1.0043x over previous
import os

import jax
import jax.numpy as jnp
from jax import lax
from jax.experimental import pallas as pl
from jax.experimental.pallas import tpu as pltpu

N_DEV = 4
SQ = 256
D = 1024
SKV = 4096
H_PER = 8
DH = 128
SCALE = 0.08838834764831843

BF16 = jnp.bfloat16
F32 = jnp.float32

PROBE = os.environ.get("KERNEL_PROBE", "")


def kernel(x, Wq, Wo, K_ext, V_ext):
    def body(x_ref, wq_ref, wo_ref, k_hbm, v_hbm, out_ref,
             xall, wqb, wob, kbuf, vbuf, obuf, ps_buf, pr_buf,
             ag_send, ag_recv, ps_send, pr_recv, kv_sems):
        i = lax.axis_index("i")

        barrier_sem = pltpu.get_barrier_semaphore()
        for d in range(1, N_DEV):
            pl.semaphore_signal(
                barrier_sem, inc=1,
                device_id=((i + d) % N_DEV,),
                device_id_type=pl.DeviceIdType.MESH,
            )
        pl.semaphore_wait(barrier_sem, N_DEV - 1)

        xall[0] = x_ref[0].astype(BF16)
        ag_rdmas = []
        for d in range(1, N_DEV):
            r = pltpu.make_async_remote_copy(
                src_ref=xall.at[0],
                dst_ref=xall.at[N_DEV - d],
                send_sem=ag_send.at[d - 1],
                recv_sem=ag_recv.at[N_DEV - d],
                device_id=((i + d) % N_DEV,),
                device_id_type=pl.DeviceIdType.MESH,
            )
            r.start()
            ag_rdmas.append(r)

        wqb[...] = wq_ref[...].astype(BF16)
        wob[...] = wo_ref[...].astype(BF16)

        h0c = i * (H_PER * DH)

        k2 = k_hbm.reshape(N_DEV, SKV, 32 * DH)
        v2 = v_hbm.reshape(N_DEV, SKV, 32 * DH)

        HPC = 2
        CPB = H_PER // HPC
        GW = HPC * DH
        N_CHUNK = CPB * N_DEV
        RING = 3

        def start_kv(c):
            d, g = divmod(c, CPB)
            b = (i + d) % N_DEV
            slot = c % RING
            ck = pltpu.make_async_copy(
                k2.at[b, :, pl.ds(h0c + g * GW, GW)], kbuf.at[slot],
                kv_sems.at[slot, 0])
            cv = pltpu.make_async_copy(
                v2.at[b, :, pl.ds(h0c + g * GW, GW)], vbuf.at[slot],
                kv_sems.at[slot, 1])
            ck.start()
            cv.start()
            return ck, cv

        if PROBE == "nodma":
            kv_inflight = {}
        else:
            kv_inflight = {c: start_kv(c) for c in range(RING)}

        send_rdmas = []
        acc = None
        for d in range(N_DEV):
            b = (i + d) % N_DEV

            if d > 0:
                recv = pltpu.make_async_remote_copy(
                    src_ref=xall.at[0], dst_ref=xall.at[d],
                    send_sem=ag_send.at[0], recv_sem=ag_recv.at[d],
                    device_id=(i,), device_id_type=pl.DeviceIdType.MESH,
                )
                recv.wait_recv()

            q = jnp.dot(xall[d], wqb[...], preferred_element_type=F32)
            q = q.astype(BF16)

            for g in range(CPB):
                c = CPB * d + g
                slot = c % RING
                if PROBE != "nodma":
                    ck, cv = kv_inflight.pop(c)
                    ck.wait()
                    cv.wait()
                for h in range(HPC):
                    hh = g * HPC + h
                    qh = q[:, hh * DH:(hh + 1) * DH]
                    if PROBE == "nocompute":
                        obuf[:, hh * DH:(hh + 1) * DH] = (
                            kbuf[slot, :SQ, h * DH:(h + 1) * DH]
                            + vbuf[slot, :SQ, h * DH:(h + 1) * DH]
                        ).astype(BF16)
                    else:
                        kh = kbuf[slot, :, h * DH:(h + 1) * DH].astype(BF16)
                        s = lax.dot_general(
                            qh, kh, (((1,), (1,)), ((), ())),
                            preferred_element_type=F32) * SCALE
                        m = jnp.max(s, axis=1, keepdims=True)
                        p = jnp.exp(s - m)
                        l = jnp.sum(p, axis=1, keepdims=True)
                        vh = vbuf[slot, :, h * DH:(h + 1) * DH].astype(BF16)
                        oh = lax.dot_general(
                            p.astype(BF16), vh, (((1,), (0,)), ((), ())),
                            preferred_element_type=F32)
                        obuf[:, hh * DH:(hh + 1) * DH] = (
                            oh / l).astype(BF16)

                if PROBE != "nodma" and c + RING < N_CHUNK:
                    kv_inflight[c + RING] = start_kv(c + RING)

            part = jnp.dot(obuf[...], wob[...], preferred_element_type=F32)

            if d == 0:
                acc = part
            else:
                ps_buf[d - 1] = part.astype(BF16)
                r = pltpu.make_async_remote_copy(
                    src_ref=ps_buf.at[d - 1],
                    dst_ref=pr_buf.at[N_DEV - d],
                    send_sem=ps_send.at[d - 1],
                    recv_sem=pr_recv.at[N_DEV - d],
                    device_id=(b,), device_id_type=pl.DeviceIdType.MESH,
                )
                r.start()
                send_rdmas.append(r)

        for s in range(1, N_DEV):
            recv = pltpu.make_async_remote_copy(
                src_ref=ps_buf.at[0], dst_ref=pr_buf.at[s],
                send_sem=ps_send.at[0], recv_sem=pr_recv.at[s],
                device_id=(i,), device_id_type=pl.DeviceIdType.MESH,
            )
            recv.wait_recv()
            acc = acc + pr_buf[s].astype(F32)

        for r in ag_rdmas:
            r.wait_send()
        for r in send_rdmas:
            r.wait_send()

        out_ref[0] = acc

    return pl.pallas_call(
        body,
        out_shape=jax.ShapeDtypeStruct((1, SQ, D), F32),
        in_specs=[
            pl.BlockSpec(memory_space=pltpu.VMEM),
            pl.BlockSpec(memory_space=pltpu.VMEM),
            pl.BlockSpec(memory_space=pltpu.VMEM),
            pl.BlockSpec(memory_space=pl.ANY),
            pl.BlockSpec(memory_space=pl.ANY),
        ],
        out_specs=pl.BlockSpec(memory_space=pltpu.VMEM),
        scratch_shapes=[
            pltpu.VMEM((N_DEV, SQ, D), BF16),
            pltpu.VMEM((D, D), BF16),
            pltpu.VMEM((D, D), BF16),
            pltpu.VMEM((3, SKV, 2 * DH), F32),
            pltpu.VMEM((3, SKV, 2 * DH), F32),
            pltpu.VMEM((SQ, D), BF16),
            pltpu.VMEM((N_DEV - 1, SQ, D), BF16),
            pltpu.VMEM((N_DEV, SQ, D), BF16),
            pltpu.SemaphoreType.DMA((N_DEV - 1,)),
            pltpu.SemaphoreType.DMA((N_DEV,)),
            pltpu.SemaphoreType.DMA((N_DEV - 1,)),
            pltpu.SemaphoreType.DMA((N_DEV,)),
            pltpu.SemaphoreType.DMA((3, 2)),
        ],
        compiler_params=pltpu.CompilerParams(
            collective_id=0,
            vmem_limit_bytes=63 * 1024 * 1024,
        ),
        interpret=(
            pltpu.InterpretParams()
            if os.environ.get("PL_INTERPRET") == "1"
            else False
        ),
    )(x, Wq, Wo, K_ext, V_ext)
